# Initial kernel scaffold; baseline (speedup 1.0000x reference)
import jax
import jax.numpy as jnp
from jax.experimental import pallas as pl


def kernel(x, face_types, edge_index, edge_attr, batch, node_mask, face_emb, W_node, b_node, W_edge, b_edge, g0_W, g0_as, g0_ad, g0_We, g0_ae, g0_b, g1_W, g1_as, g1_ad, g1_We, g1_ae, g1_b, g2_W, g2_as, g2_ad, g2_We, g2_ae, g2_b, W_mu, b_mu, W_lv, b_lv):
    raise NotImplementedError("write your pallas kernel here")


# speedup vs baseline: 1.0005x; 1.0005x over previous
import jax
import jax.numpy as jnp
from jax.experimental import pallas as pl
from jax.experimental.pallas import tpu as pltpu

N = 50000
H = 4
C = 16
HID = 64
B = 16
NEG_SLOPE = 0.2


def _gat_xla(h, eh, src, dst, W, a_s, a_d, We, a_e, b):
    xs = (h @ W).reshape(-1, H, C)
    ehp = (eh @ We).reshape(-1, H, C)
    al_s = (xs * a_s[None, :, :]).sum(-1)
    al_d = (xs * a_d[None, :, :]).sum(-1)
    al_e = (ehp * a_e[None, :, :]).sum(-1)
    alpha = al_s[src] + al_d[dst] + al_e
    alpha = jnp.where(alpha >= 0, alpha, NEG_SLOPE * alpha)
    amax = jax.ops.segment_max(alpha, dst, num_segments=N)
    amax = jnp.where(jnp.isfinite(amax), amax, 0.0)
    ex = jnp.exp(alpha - amax[dst])
    den = jax.ops.segment_sum(ex, dst, num_segments=N)
    att = ex / (den[dst] + 1e-16)
    msg = xs[src] * att[:, :, None]
    out = jax.ops.segment_sum(msg, dst, num_segments=N).reshape(-1, H * C) + b
    return out


def _pool_proj_kernel(h_ref, batch_ref, mask_ref, wmu_ref, bmu_ref, wlv_ref,
                      blv_ref, mu_ref, lv_ref, acc_ref, cnt_ref):
    i = pl.program_id(0)
    nblk = pl.num_programs(0)

    @pl.when(i == 0)
    def _():
        acc_ref[...] = jnp.zeros_like(acc_ref)
        cnt_ref[...] = jnp.zeros_like(cnt_ref)

    hm = h_ref[...] * mask_ref[...]
    onehot = (batch_ref[...][:, 0][None, :] == jax.lax.broadcasted_iota(
        jnp.int32, (B, h_ref.shape[0]), 0)).astype(jnp.float32)
    acc_ref[...] += jnp.dot(onehot, hm, preferred_element_type=jnp.float32)
    cnt_ref[...] += jnp.dot(onehot, mask_ref[...],
                            preferred_element_type=jnp.float32)

    @pl.when(i == nblk - 1)
    def _():
        hp = acc_ref[...] / jnp.clip(cnt_ref[...], 1.0, None)
        mu_ref[...] = jnp.dot(hp, wmu_ref[...],
                              preferred_element_type=jnp.float32) + bmu_ref[...][None, :]
        lv_ref[...] = jnp.dot(hp, wlv_ref[...],
                              preferred_element_type=jnp.float32) + blv_ref[...][None, :]


def _pool_proj(h, batch, node_mask, W_mu, b_mu, W_lv, b_lv):
    BLK = 2000
    nblk = N // BLK
    lat = W_mu.shape[1]
    return pl.pallas_call(
        _pool_proj_kernel,
        grid=(nblk,),
        in_specs=[
            pl.BlockSpec((BLK, HID), lambda i: (i, 0)),
            pl.BlockSpec((BLK, 1), lambda i: (i, 0)),
            pl.BlockSpec((BLK, 1), lambda i: (i, 0)),
            pl.BlockSpec((HID, lat), lambda i: (0, 0)),
            pl.BlockSpec((lat,), lambda i: (0,)),
            pl.BlockSpec((HID, lat), lambda i: (0, 0)),
            pl.BlockSpec((lat,), lambda i: (0,)),
        ],
        out_specs=[
            pl.BlockSpec((B, lat), lambda i: (0, 0)),
            pl.BlockSpec((B, lat), lambda i: (0, 0)),
        ],
        out_shape=[
            jax.ShapeDtypeStruct((B, lat), jnp.float32),
            jax.ShapeDtypeStruct((B, lat), jnp.float32),
        ],
        scratch_shapes=[
            pltpu.VMEM((B, HID), jnp.float32),
            pltpu.VMEM((B, 1), jnp.float32),
        ],
    )(h, batch[:, None], node_mask[:, None], W_mu, b_mu, W_lv, b_lv)


def kernel(x, face_types, edge_index, edge_attr, batch, node_mask, face_emb,
           W_node, b_node, W_edge, b_edge,
           g0_W, g0_as, g0_ad, g0_We, g0_ae, g0_b,
           g1_W, g1_as, g1_ad, g1_We, g1_ae, g1_b,
           g2_W, g2_as, g2_ad, g2_We, g2_ae, g2_b,
           W_mu, b_mu, W_lv, b_lv):
    fe = jnp.take(face_emb, face_types, axis=0)
    h = jnp.concatenate([x, fe], axis=-1)
    h = jax.nn.relu(h @ W_node + b_node)
    eh = jax.nn.relu(edge_attr @ W_edge + b_edge)
    src = edge_index[0]
    dst = edge_index[1]
    for (W, a_s, a_d, We, a_e, b) in (
            (g0_W, g0_as, g0_ad, g0_We, g0_ae, g0_b),
            (g1_W, g1_as, g1_ad, g1_We, g1_ae, g1_b),
            (g2_W, g2_as, g2_ad, g2_We, g2_ae, g2_b)):
        h = jax.nn.relu(_gat_xla(h, eh, src, dst, W, a_s, a_d, We, a_e, b))
    return _pool_proj(h, batch, node_mask, W_mu, b_mu, W_lv, b_lv)


# speedup vs baseline: 1.0006x; 1.0000x over previous
import jax
import jax.numpy as jnp
from jax import lax
from jax.experimental import pallas as pl
from jax.experimental.pallas import tpu as pltpu

N = 50000
H = 4
C = 16
HID = 64
B = 16
LAT = 32
NEG = 0.2
NBLK = 2000


def _gat_xla(h, eh, src, dst, W, a_s, a_d, We, a_e, b):
    xs = (h @ W).reshape(-1, H, C)
    ehp = (eh @ We).reshape(-1, H, C)
    al_s = (xs * a_s[None, :, :]).sum(-1)
    al_d = (xs * a_d[None, :, :]).sum(-1)
    al_e = (ehp * a_e[None, :, :]).sum(-1)
    alpha = al_s[src] + al_d[dst] + al_e
    alpha = jnp.where(alpha >= 0, alpha, NEG * alpha)
    amax = jax.ops.segment_max(alpha, dst, num_segments=N)
    amax = jnp.where(jnp.isfinite(amax), amax, 0.0)
    ex = jnp.exp(alpha - amax[dst])
    den = jax.ops.segment_sum(ex, dst, num_segments=N)
    att = ex / (den[dst] + 1e-16)
    msg = xs[src] * att[:, :, None]
    out = jax.ops.segment_sum(msg, dst, num_segments=N).reshape(-1, H * C) + b
    return out


def _pool_proj_kernel(h_ref, batch_ref, mask_ref, wmu_ref, bmu_ref, wlv_ref,
                      blv_ref, mu_ref, lv_ref, acc_ref, cnt_ref):
    i = pl.program_id(0)
    nblk = pl.num_programs(0)

    @pl.when(i == 0)
    def _():
        acc_ref[...] = jnp.zeros_like(acc_ref)
        cnt_ref[...] = jnp.zeros_like(cnt_ref)

    hm = h_ref[...] * mask_ref[...]
    onehot = (batch_ref[...][:, 0][None, :] == lax.broadcasted_iota(
        jnp.int32, (B, h_ref.shape[0]), 0)).astype(jnp.float32)
    acc_ref[...] += jnp.dot(onehot, hm, preferred_element_type=jnp.float32)
    cnt_ref[...] += jnp.dot(onehot, mask_ref[...],
                            preferred_element_type=jnp.float32)

    @pl.when(i == nblk - 1)
    def _():
        hp = acc_ref[...] / jnp.clip(cnt_ref[...], 1.0, None)
        mu_ref[...] = jnp.dot(hp, wmu_ref[...],
                              preferred_element_type=jnp.float32) + bmu_ref[...][None, :]
        lv_ref[...] = jnp.dot(hp, wlv_ref[...],
                              preferred_element_type=jnp.float32) + blv_ref[...][None, :]


def _pool_proj(h, batch, node_mask, W_mu, b_mu, W_lv, b_lv):
    nblk = N // NBLK
    lat = W_mu.shape[1]
    return pl.pallas_call(
        _pool_proj_kernel,
        grid=(nblk,),
        in_specs=[
            pl.BlockSpec((NBLK, HID), lambda i: (i, 0)),
            pl.BlockSpec((NBLK, 1), lambda i: (i, 0)),
            pl.BlockSpec((NBLK, 1), lambda i: (i, 0)),
            pl.BlockSpec((HID, lat), lambda i: (0, 0)),
            pl.BlockSpec((lat,), lambda i: (0,)),
            pl.BlockSpec((HID, lat), lambda i: (0, 0)),
            pl.BlockSpec((lat,), lambda i: (0,)),
        ],
        out_specs=[
            pl.BlockSpec((B, lat), lambda i: (0, 0)),
            pl.BlockSpec((B, lat), lambda i: (0, 0)),
        ],
        out_shape=[
            jax.ShapeDtypeStruct((B, lat), jnp.float32),
            jax.ShapeDtypeStruct((B, lat), jnp.float32),
        ],
        scratch_shapes=[
            pltpu.VMEM((B, HID), jnp.float32),
            pltpu.VMEM((B, 1), jnp.float32),
        ],
    )(h, batch[:, None], node_mask[:, None], W_mu, b_mu, W_lv, b_lv)


def kernel(x, face_types, edge_index, edge_attr, batch, node_mask, face_emb,
           W_node, b_node, W_edge, b_edge,
           g0_W, g0_as, g0_ad, g0_We, g0_ae, g0_b,
           g1_W, g1_as, g1_ad, g1_We, g1_ae, g1_b,
           g2_W, g2_as, g2_ad, g2_We, g2_ae, g2_b,
           W_mu, b_mu, W_lv, b_lv):
    fe = jnp.take(face_emb, face_types, axis=0)
    h = jnp.concatenate([x, fe], axis=-1)
    h = jax.nn.relu(h @ W_node + b_node)
    eh = jax.nn.relu(edge_attr @ W_edge + b_edge)
    src = edge_index[0]
    dst = edge_index[1]
    for (W, a_s, a_d, We, a_e, b) in (
            (g0_W, g0_as, g0_ad, g0_We, g0_ae, g0_b),
            (g1_W, g1_as, g1_ad, g1_We, g1_ae, g1_b),
            (g2_W, g2_as, g2_ad, g2_We, g2_ae, g2_b)):
        h = jax.nn.relu(_gat_xla(h, eh, src, dst, W, a_s, a_d, We, a_e, b))
    return _pool_proj(h, batch, node_mask, W_mu, b_mu, W_lv, b_lv)
